# baseline (speedup 1.0000x reference)
import functools

import jax
import jax.numpy as jnp
from jax import lax
from jax.experimental import pallas as pl
from jax.experimental.pallas import tpu as pltpu
from jax.experimental.pallas import tpu_sc as plsc

_ROW = 256
_FRAME_ROWS = 256
_NW = 4
_RPW = _FRAME_ROWS // _NW


def _sc_gather(table, row_idx):
    mesh = plsc.VectorSubcoreMesh(
        core_axis_name="c", subcore_axis_name="s", num_cores=1, num_subcores=4
    )

    @functools.partial(
        pl.kernel,
        mesh=mesh,
        out_type=jax.ShapeDtypeStruct((_FRAME_ROWS, _ROW), jnp.float32),
        scratch_types=[
            pltpu.VMEM((_RPW,), jnp.int32),
            pltpu.VMEM((_RPW, _ROW), jnp.float32),
            pltpu.SemaphoreType.DMA,
        ],
    )
    def k(table_hbm, idx_hbm, out_hbm, idx_v, rows_v, sem):
        wid = lax.axis_index("s")
        pltpu.sync_copy(idx_hbm.at[wid], idx_v)
        pltpu.async_copy(table_hbm.at[idx_v], rows_v, sem).wait()
        pltpu.sync_copy(rows_v, out_hbm.at[pl.ds(wid * _RPW, _RPW)])

    return k(table, row_idx)


def kernel(noise, idx):
    length = noise.shape[0]
    table = noise.reshape(length * _FRAME_ROWS, _ROW)
    base = (jnp.asarray(idx, jnp.int32) % length) * _FRAME_ROWS
    row_idx = (base + jnp.arange(_FRAME_ROWS, dtype=jnp.int32)).reshape(
        _NW, _RPW
    )
    out = _sc_gather(table, row_idx)
    return out.reshape(1, *noise.shape[1:])


# speedup vs baseline: 1.0489x; 1.0489x over previous
import functools

import jax
import jax.numpy as jnp
from jax import lax
from jax.experimental import pallas as pl
from jax.experimental.pallas import tpu as pltpu
from jax.experimental.pallas import tpu_sc as plsc

_ROW = 256
_FRAME_ROWS = 256
_NW = 16
_RPW = _FRAME_ROWS // _NW


def _sc_gather(table, row_idx):
    mesh = plsc.VectorSubcoreMesh(
        core_axis_name="c", subcore_axis_name="s", num_cores=1
    )

    @functools.partial(
        pl.kernel,
        mesh=mesh,
        out_type=jax.ShapeDtypeStruct((_FRAME_ROWS, _ROW), jnp.float32),
        scratch_types=[
            pltpu.VMEM((_RPW,), jnp.int32),
            pltpu.VMEM((_RPW, _ROW), jnp.float32),
            pltpu.SemaphoreType.DMA,
        ],
    )
    def k(table_hbm, idx_hbm, out_hbm, idx_v, rows_v, sem):
        wid = lax.axis_index("s")
        pltpu.sync_copy(idx_hbm.at[wid], idx_v)
        pltpu.async_copy(table_hbm.at[idx_v], rows_v, sem).wait()
        pltpu.sync_copy(rows_v, out_hbm.at[pl.ds(wid * _RPW, _RPW)])

    return k(table, row_idx)


def kernel(noise, idx):
    length = noise.shape[0]
    table = noise.reshape(length * _FRAME_ROWS, _ROW)
    base = (jnp.asarray(idx, jnp.int32) % length) * _FRAME_ROWS
    row_idx = (base + jnp.arange(_FRAME_ROWS, dtype=jnp.int32)).reshape(
        _NW, _RPW
    )
    out = _sc_gather(table, row_idx)
    return out.reshape(1, *noise.shape[1:])
